# baseline (speedup 1.0000x reference)
import functools

import jax
import jax.numpy as jnp
from jax.experimental import pallas as pl
from jax.experimental.pallas import tpu as pltpu

N = 4096
D = 128
CH = 256
K = 4
NCH = N // CH


def _gcn_kernel(x_ref, w_ref, adj_hbm, gamma_ref, beta_ref, out_ref,
                hproj_ref, buf_ref, sem):
    t = pl.program_id(0)

    def copy_in(c, slot):
        return pltpu.make_async_copy(
            adj_hbm.at[t, pl.ds(c * CH, CH), :],
            buf_ref.at[slot],
            sem.at[slot],
        )

    for k in range(K):
        copy_in(k, k).start()

    hproj_ref[...] = jax.lax.dot_general(
        x_ref[0], w_ref[0],
        dimension_numbers=(((1,), (1,)), ((), ())),
        preferred_element_type=jnp.float32,
    ).astype(jnp.bfloat16)

    gamma = gamma_ref[0]
    beta = beta_ref[0]

    def body(c, _):
        slot = jax.lax.rem(c, K)
        copy_in(c, slot).wait()
        agg = jnp.dot(buf_ref[slot].astype(jnp.bfloat16), hproj_ref[...],
                      preferred_element_type=jnp.float32)
        h = agg + x_ref[0, pl.ds(c * CH, CH), :]
        mu = jnp.mean(h, axis=-1, keepdims=True)
        cen = h - mu
        var = jnp.mean(cen * cen, axis=-1, keepdims=True)
        out_ref[0, pl.ds(c * CH, CH), :] = (
            cen * jax.lax.rsqrt(var + 1e-5) * gamma + beta)

        @pl.when(c + K < NCH)
        def _():
            copy_in(c + K, slot).start()

        return 0

    jax.lax.fori_loop(0, NCH, body, 0)


@jax.jit
def _gcn(node_feats, adj_dict, Ws, gammas, betas):
    out = pl.pallas_call(
        _gcn_kernel,
        grid=(2,),
        in_specs=[
            pl.BlockSpec((1, N, D), lambda t: (t, 0, 0)),
            pl.BlockSpec((1, D, D), lambda t: (t, 0, 0)),
            pl.BlockSpec(memory_space=pl.ANY),
            pl.BlockSpec((1, 1, D), lambda t: (t, 0, 0)),
            pl.BlockSpec((1, 1, D), lambda t: (t, 0, 0)),
        ],
        out_specs=pl.BlockSpec((1, N, D), lambda t: (t, 0, 0)),
        out_shape=jax.ShapeDtypeStruct((2, N, D), jnp.float32),
        scratch_shapes=[
            pltpu.VMEM((N, D), jnp.bfloat16),
            pltpu.VMEM((K, CH, N), jnp.float32),
            pltpu.SemaphoreType.DMA((K,)),
        ],
        compiler_params=pltpu.CompilerParams(
            dimension_semantics=("parallel",),
        ),
    )(node_feats, Ws, adj_dict, gammas, betas)
    return out.reshape(2 * N, D)


def kernel(node_feats, adj_dict, W0, W1, gamma0, beta0, gamma1, beta1):
    Ws = jnp.stack((W0, W1))
    gammas = jnp.stack((gamma0, gamma1)).reshape(2, 1, D)
    betas = jnp.stack((beta0, beta1)).reshape(2, 1, D)
    return _gcn(node_feats, adj_dict, Ws, gammas, betas)
